# Initial kernel scaffold; baseline (speedup 1.0000x reference)
import jax
import jax.numpy as jnp
from jax.experimental import pallas as pl


def kernel(x, edge_index, edge_attr, Wih, Whh, bih, bhh, W1, b1, W2, b2, Wfc, bfc):
    raise NotImplementedError("write your pallas kernel here")


# speedup vs baseline: 2.7370x; 2.7370x over previous
import functools

import jax
import jax.numpy as jnp
from jax import lax
from jax.experimental import pallas as pl
from jax.experimental.pallas import tpu as pltpu
from jax.experimental.pallas import tpu_sc as plsc

N = 50000
T = 20
H = 64
E = 800000

_B = 2000
_GRID = N // _B


def _gru_fold_body(x_ref, wih_ref, whhT_ref, bih_ref, bhh_ref, w1_ref,
                   d0_ref, d1_ref, outA_ref, outB_ref):
    wih = wih_ref[...]
    whhT = whhT_ref[...]
    bih = bih_ref[...]
    bhh = bhh_ref[...]
    h = jnp.zeros((_B, H), dtype=jnp.float32)
    for t in range(T):
        xt = x_ref[:, t:t + 1]
        gi = xt * wih + bih
        gh = jnp.dot(h, whhT, preferred_element_type=jnp.float32) + bhh
        r = jax.nn.sigmoid(gi[:, :H] + gh[:, :H])
        z = jax.nn.sigmoid(gi[:, H:2 * H] + gh[:, H:2 * H])
        n = jnp.tanh(gi[:, 2 * H:] + r * gh[:, 2 * H:])
        h = (1.0 - z) * n + z * h
    dinv = lax.rsqrt(1.0 + d0_ref[...] + d1_ref[...])
    xwp = dinv * jnp.dot(h, w1_ref[...], preferred_element_type=jnp.float32)
    outA_ref[...] = xwp[:, :32]
    outB_ref[...] = xwp[:, 32:]


def _gru_fold(x, wihT, whhT, bihT, bhhT, W1, d0, d1):
    spec_rows = lambda c: pl.BlockSpec((_B, c), lambda i: (i, 0))
    spec_full = lambda r, c: pl.BlockSpec((r, c), lambda i: (0, 0))
    return pl.pallas_call(
        _gru_fold_body,
        grid=(_GRID,),
        in_specs=[
            spec_rows(T),
            spec_full(1, 3 * H), spec_full(H, 3 * H),
            spec_full(1, 3 * H), spec_full(1, 3 * H),
            spec_full(H, H),
            spec_rows(1), spec_rows(1),
        ],
        out_specs=[spec_rows(32), spec_rows(32)],
        out_shape=[jax.ShapeDtypeStruct((N, 32), jnp.float32)] * 2,
    )(x, wihT, whhT, bihT, bhhT, W1, d0, d1)


def _combine_mid_body(sA_ref, sB_ref, xA_ref, xB_ref, b_ref, w_ref,
                      d0_ref, d1_ref, outA_ref, outB_ref):
    dinv = lax.rsqrt(1.0 + d0_ref[...] + d1_ref[...])
    s = jnp.concatenate([sA_ref[...], sB_ref[...]], axis=1)
    xp = jnp.concatenate([xA_ref[...], xB_ref[...]], axis=1)
    h = jax.nn.relu(dinv * (s + xp) + b_ref[...])
    xwp = dinv * jnp.dot(h, w_ref[...], preferred_element_type=jnp.float32)
    outA_ref[...] = xwp[:, :32]
    outB_ref[...] = xwp[:, 32:]


def _combine_mid(sA, sB, xA, xB, b, W, d0, d1):
    spec_rows = lambda c: pl.BlockSpec((_B, c), lambda i: (i, 0))
    spec_full = lambda r, c: pl.BlockSpec((r, c), lambda i: (0, 0))
    return pl.pallas_call(
        _combine_mid_body,
        grid=(_GRID,),
        in_specs=[
            spec_rows(32), spec_rows(32), spec_rows(32), spec_rows(32),
            spec_full(1, H), spec_full(H, H),
            spec_rows(1), spec_rows(1),
        ],
        out_specs=[spec_rows(32), spec_rows(32)],
        out_shape=[jax.ShapeDtypeStruct((N, 32), jnp.float32)] * 2,
    )(sA, sB, xA, xB, b, W, d0, d1)


def _combine_final_body(sA_ref, sB_ref, xA_ref, xB_ref, b_ref, wfc_ref,
                        bfc_ref, d0_ref, d1_ref, out_ref):
    dinv = lax.rsqrt(1.0 + d0_ref[...] + d1_ref[...])
    s = jnp.concatenate([sA_ref[...], sB_ref[...]], axis=1)
    xp = jnp.concatenate([xA_ref[...], xB_ref[...]], axis=1)
    h = jax.nn.relu(dinv * (s + xp) + b_ref[...])
    out_ref[...] = (jnp.dot(h, wfc_ref[...], preferred_element_type=jnp.float32)
                    + bfc_ref[...])


def _combine_final(sA, sB, xA, xB, b, Wfc, bfc, d0, d1):
    spec_rows = lambda c: pl.BlockSpec((_B, c), lambda i: (i, 0))
    spec_full = lambda r, c: pl.BlockSpec((r, c), lambda i: (0, 0))
    return pl.pallas_call(
        _combine_final_body,
        grid=(_GRID,),
        in_specs=[
            spec_rows(32), spec_rows(32), spec_rows(32), spec_rows(32),
            spec_full(1, H), spec_full(H, H), spec_full(1, H),
            spec_rows(1), spec_rows(1),
        ],
        out_specs=spec_rows(H),
        out_shape=jax.ShapeDtypeStruct((N, H), jnp.float32),
    )(sA, sB, xA, xB, b, Wfc, bfc, d0, d1)


def _degree(col, ew):
    d = jnp.zeros((N,), jnp.float32).at[col].add(ew)
    return d.reshape(N, 1), jnp.zeros((N, 1), jnp.float32)


def _spmm(xA, xB, row, col, ew):
    xw = jnp.concatenate([xA, xB], axis=1)
    msg = xw[row] * ew[:, None]
    s = jnp.zeros((N, H), jnp.float32).at[col].add(msg)
    return s[:, :32], s[:, 32:]


def kernel(x, edge_index, edge_attr, Wih, Whh, bih, bhh, W1, b1, W2, b2,
           Wfc, bfc):
    row = edge_index[0].astype(jnp.int32)
    col = edge_index[1].astype(jnp.int32)
    ew = edge_attr[:, 0]

    wihT = Wih.reshape(1, 3 * H)
    whhT = Whh.T
    bihT = bih.reshape(1, 3 * H)
    bhhT = bhh.reshape(1, 3 * H)
    b1r = b1.reshape(1, H)
    b2r = b2.reshape(1, H)
    bfcr = bfc.reshape(1, H)

    d0, d1 = _degree(col, ew)
    x1A, x1B = _gru_fold(x, wihT, whhT, bihT, bhhT, W1, d0, d1)
    s1A, s1B = _spmm(x1A, x1B, row, col, ew)
    x2A, x2B = _combine_mid(s1A, s1B, x1A, x1B, b1r, W2, d0, d1)
    s2A, s2B = _spmm(x2A, x2B, row, col, ew)
    return _combine_final(s2A, s2B, x2A, x2B, b2r, Wfc, bfcr, d0, d1)
